# baseline (speedup 1.0000x reference)
import functools

import jax
import jax.numpy as jnp
from jax import lax
from jax.experimental import pallas as pl
from jax.experimental.pallas import tpu as pltpu
from jax.experimental.pallas import tpu_sc as plsc

NUM_CORES = 2
NUM_SUBCORES = 16
LANES = 16
NUM_WORKERS = NUM_CORES * NUM_SUBCORES

BATCH = 16384
D = 64
BPW = BATCH // NUM_WORKERS
CHUNK = 128
NCHUNK = BPW // CHUNK
GROUPS = CHUNK // LANES

_mesh = plsc.VectorSubcoreMesh(core_axis_name="c", subcore_axis_name="s")


@functools.partial(
    pl.kernel,
    out_type=jax.ShapeDtypeStruct((BATCH,), jnp.float32),
    mesh=_mesh,
    compiler_params=pltpu.CompilerParams(needs_layout_passes=False, use_tc_tiling_on_sc=False),
    scratch_types=[
        pltpu.VMEM((NCHUNK, CHUNK), jnp.int32),
        pltpu.VMEM((NCHUNK, CHUNK), jnp.int32),
        pltpu.VMEM((2, CHUNK, D), jnp.float32),
        pltpu.VMEM((2, CHUNK, D), jnp.float32),
        pltpu.VMEM((D,), jnp.float32),
        pltpu.VMEM((BPW,), jnp.float32),
        pltpu.SemaphoreType.DMA,
        pltpu.SemaphoreType.DMA,
    ],
)
def _gmf_kernel(users_hbm, movies_hbm, ut_hbm, mt_hbm, w_hbm, out_hbm,
                uidx_v, midx_v, urows_v, mrows_v, w_v, out_v, sem_a, sem_b):
    wid = lax.axis_index("s") * NUM_CORES + lax.axis_index("c")
    base = wid * BPW

    pltpu.sync_copy(w_hbm, w_v)
    for c in range(NCHUNK):
        pltpu.sync_copy(users_hbm.at[pl.ds(base + c * CHUNK, CHUNK)], uidx_v.at[c])
        pltpu.sync_copy(movies_hbm.at[pl.ds(base + c * CHUNK, CHUNK)], midx_v.at[c])

    wvecs = [w_v[pl.ds(k * LANES, LANES)] for k in range(D // LANES)]
    ws = [wvecs[d // LANES][d % LANES] for d in range(D)]

    sems = [sem_a, sem_b]

    def start_gather(c):
        buf = c % 2
        pltpu.async_copy(ut_hbm.at[uidx_v.at[c]], urows_v.at[buf], sems[buf])
        pltpu.async_copy(mt_hbm.at[midx_v.at[c]], mrows_v.at[buf], sems[buf])

    def wait_gather(c):
        buf = c % 2
        pltpu.make_async_copy(ut_hbm.at[uidx_v.at[c]], urows_v.at[buf], sems[buf]).wait()
        pltpu.make_async_copy(mt_hbm.at[midx_v.at[c]], mrows_v.at[buf], sems[buf]).wait()

    start_gather(0)
    for c in range(NCHUNK):
        if c + 1 < NCHUNK:
            start_gather(c + 1)
        wait_gather(c)
        buf = c % 2
        u_buf = urows_v.at[buf]
        m_buf = mrows_v.at[buf]

        def group_body(g, carry, u_buf=u_buf, m_buf=m_buf, c=c):
            rows = lax.iota(jnp.int32, LANES) + g * LANES
            acc = jnp.zeros((LANES,), jnp.float32)
            for d in range(D):
                dv = jnp.full((LANES,), d, jnp.int32)
                uv = plsc.load_gather(u_buf, [rows, dv])
                mv = plsc.load_gather(m_buf, [rows, dv])
                acc = acc + uv * mv * ws[d]
            out_v[pl.ds(c * CHUNK + g * LANES, LANES)] = acc
            return carry

        lax.fori_loop(0, GROUPS, group_body, 0)

    pltpu.sync_copy(out_v, out_hbm.at[pl.ds(base, BPW)])


def kernel(users, movies, user_table, movie_table, W):
    out = _gmf_kernel(users, movies, user_table, movie_table, W.reshape(D))
    return out.reshape(BATCH, 1)
